# baseline (speedup 1.0000x reference)
import functools

import jax
import jax.numpy as jnp
from jax import lax
from jax.experimental import pallas as pl
from jax.experimental.pallas import tpu as pltpu
from jax.experimental.pallas import tpu_sc as plsc

N = 10000
NNZ = 160000
NE = 5000
D = 256

NC = 2
NS = 16
LANES = 16

DH = D // NC
NEP = 6144
NP = 10240
EPC = NEP // NS
VPC = NP // NS
CH = 128
CPS = 80
NNZP = NS * CPS * CH
HB = CPS // 2
ST = 128
WO = 128


def _sc_gather_scatter(xs, vtx2, edg2, deg_e_pad):
    mesh = plsc.VectorSubcoreMesh(core_axis_name="c", subcore_axis_name="s")

    @functools.partial(
        pl.kernel,
        out_type=(
            jax.ShapeDtypeStruct((NC, NP, DH), jnp.float32),
            jax.ShapeDtypeStruct((NC, NEP, DH), jnp.float32),
        ),
        mesh=mesh,
        scratch_types=[
            pltpu.VMEM_SHARED((NP, DH), jnp.float32),
            pltpu.VMEM_SHARED((NEP,), jnp.float32),
            pltpu.VMEM((HB, CH), jnp.int32),
            pltpu.VMEM((HB, CH), jnp.int32),
            pltpu.VMEM((ST, DH), jnp.float32),
            pltpu.VMEM((ST, DH), jnp.float32),
            pltpu.VMEM((ST,), jnp.float32),
            pltpu.VMEM((EPC,), jnp.float32),
            pltpu.VMEM((ST,), jnp.float32),
            pltpu.VMEM((ST,), jnp.float32),
            pltpu.VMEM((ST,), jnp.float32),
            pltpu.SemaphoreType.DMA,
            pltpu.SemaphoreType.DMA,
            pltpu.SemaphoreType.DMA,
            pltpu.SemaphoreType.DMA,
            pltpu.SemaphoreType.DMA,
        ],
        compiler_params=pltpu.CompilerParams(needs_layout_passes=False),
    )
    def body(xs_hbm, vtx_hbm, edg_hbm, dege_hbm, out_hbm, xe_hbm,
             acc_sh, cnt_sh,
             vidx, eidx, rows0, rows1, ones, zcnt, cbuf, degb, scaleb,
             sem0, sem1, sem2, sem3, sem4):
        c = lax.axis_index("c")
        s = lax.axis_index("s")
        zeros16 = jnp.zeros((LANES,), jnp.float32)
        ones16 = jnp.ones((LANES,), jnp.float32)

        def zrow(r, _):
            for cc in range(DH // LANES):
                rows0[r, pl.ds(cc * LANES, LANES)] = zeros16
            return 0
        lax.fori_loop(0, ST, zrow, 0)
        for g in range(ST // LANES):
            ones[pl.ds(g * LANES, LANES)] = ones16
        for g in range(EPC // LANES):
            zcnt[pl.ds(g * LANES, LANES)] = zeros16

        e0 = s * EPC
        v0 = s * VPC
        b0 = s * CPS

        def run_phase(gather_tab, gsel, ssel, with_cnt):
            for h in range(2):
                pltpu.sync_copy(vtx_hbm.at[pl.ds(b0 + h * HB, HB)], vidx)
                pltpu.sync_copy(edg_hbm.at[pl.ds(b0 + h * HB, HB)], eidx)
                gi = vidx if gsel == "v" else eidx
                si = vidx if ssel == "v" else eidx

                def gath(j, buf, sem):
                    pltpu.async_copy(
                        gather_tab.at[c].at[gi.at[j]], buf, sem)

                def gath_wait(j, buf, sem):
                    pltpu.make_async_copy(
                        gather_tab.at[c].at[gi.at[j]], buf, sem).wait()

                def scat(j, buf, sem):
                    pltpu.async_copy(buf, acc_sh.at[si.at[j]], sem, add=True)
                    if with_cnt:
                        pltpu.async_copy(ones, cnt_sh.at[si.at[j]],
                                         sem4, add=True)

                def scat_wait(j, buf, sem):
                    pltpu.make_async_copy(buf, acc_sh.at[si.at[j]],
                                          sem).wait()

                gath(0, rows0, sem0)
                gath(1, rows1, sem1)

                def step(j2, _):
                    j = j2 * 2
                    gath_wait(j, rows0, sem0)
                    scat(j, rows0, sem2)
                    gath_wait(j + 1, rows1, sem1)
                    scat(j + 1, rows1, sem3)

                    @pl.when(j2 < HB // 2 - 1)
                    def _():
                        scat_wait(j, rows0, sem2)
                        gath(j + 2, rows0, sem0)
                        scat_wait(j + 1, rows1, sem3)
                        gath(j + 3, rows1, sem1)
                    return 0
                lax.fori_loop(0, HB // 2, step, 0)
                scat_wait(HB - 2, rows0, sem2)
                scat_wait(HB - 1, rows1, sem3)
                if with_cnt:
                    def drain(_, __):
                        pltpu.make_async_copy(ones, cnt_sh.at[si.at[0]],
                                              sem4).wait()
                        return 0
                    lax.fori_loop(0, HB, drain, 0)

        with jax.named_scope("zero_xe"):
            for j in range(EPC // ST):
                pltpu.sync_copy(rows0, acc_sh.at[pl.ds(e0 + j * ST, ST)])
            pltpu.sync_copy(zcnt, cnt_sh.at[pl.ds(e0, EPC)])
            plsc.subcore_barrier()

        with jax.named_scope("phase1"):
            run_phase(xs_hbm, "v", "e", True)
            plsc.subcore_barrier()

        with jax.named_scope("rescale"):
          for j in range(EPC // ST):
            r0 = e0 + j * ST
            pltpu.sync_copy(acc_sh.at[pl.ds(r0, ST)], rows0)
            pltpu.sync_copy(cnt_sh.at[pl.ds(r0, ST)], cbuf)
            pltpu.sync_copy(dege_hbm.at[pl.ds(r0, ST)], degb)
            for g in range(ST // LANES):
                sl = pl.ds(g * LANES, LANES)
                scaleb[sl] = degb[sl] / jnp.maximum(cbuf[sl], 1.0)

            def scalerow(r, _):
                bs = plsc.load_gather(
                    scaleb, [jnp.full((LANES,), r, jnp.int32)])
                for cc in range(DH // LANES):
                    sl = pl.ds(cc * LANES, LANES)
                    rows0[r, sl] = rows0[r, sl] * bs
                return 0
            lax.fori_loop(0, ST, scalerow, 0)
            pltpu.sync_copy(rows0, xe_hbm.at[c].at[pl.ds(r0, ST)])
        plsc.subcore_barrier()

        with jax.named_scope("rezero"):
          def zrow2(r, _):
            for cc in range(DH // LANES):
                rows0[r, pl.ds(cc * LANES, LANES)] = zeros16
            return 0
          lax.fori_loop(0, ST, zrow2, 0)
          for j in range(VPC // WO):
            pltpu.sync_copy(rows0, acc_sh.at[pl.ds(v0 + j * WO, WO)])
          plsc.subcore_barrier()

        with jax.named_scope("phase2"):
            run_phase(xe_hbm, "e", "v", False)
            plsc.subcore_barrier()

        with jax.named_scope("writeout"):
          for j in range(VPC // WO):
            w0 = v0 + j * WO
            pltpu.sync_copy(acc_sh.at[pl.ds(w0, WO)], rows0)
            pltpu.sync_copy(rows0, out_hbm.at[c].at[pl.ds(w0, WO)])

    return body(xs, vtx2, edg2, deg_e_pad)


def _tc_tail_body(xv_ref, x0_ref, degv_ref, wt_ref, p_ref, out_ref):
    degv = degv_ref[...]
    xv = xv_ref[...] * degv
    n = jnp.sum(xv * xv, axis=1, keepdims=True)
    rn = jnp.sqrt(n)
    scale = jnp.where(rn > 0, 1.0 / rn, 0.0)
    xi = p_ref[0] * (xv * scale) + p_ref[1] * x0_ref[...]
    mm = jnp.dot(xi, wt_ref[...], preferred_element_type=jnp.float32)
    out_ref[...] = p_ref[2] * xi + p_ref[3] * mm


def _tc_tail(xv, x0, degv, wt, params):
    blk = 1000
    grid = (N // blk,)
    return pl.pallas_call(
        _tc_tail_body,
        grid=grid,
        in_specs=[
            pl.BlockSpec((blk, D), lambda i: (i, 0)),
            pl.BlockSpec((blk, D), lambda i: (i, 0)),
            pl.BlockSpec((blk, 1), lambda i: (i, 0)),
            pl.BlockSpec((D, D), lambda i: (0, 0)),
            pl.BlockSpec(memory_space=pltpu.SMEM),
        ],
        out_specs=pl.BlockSpec((blk, D), lambda i: (i, 0)),
        out_shape=jax.ShapeDtypeStruct((N, D), jnp.float32),
    )(xv, x0, degv, wt, params)


def kernel(X, vertex, edges, alpha, beta, X0, degE, degV, W):
    xs = jnp.pad(X.reshape(N, NC, DH).transpose(1, 0, 2),
                 ((0, 0), (0, NP - N), (0, 0)))
    vtx2 = jnp.concatenate(
        [vertex, jnp.full((NNZP - NNZ,), N, jnp.int32)]).reshape(NS * CPS, CH)
    edg2 = jnp.concatenate(
        [edges, jnp.full((NNZP - NNZ,), NE, jnp.int32)]).reshape(NS * CPS, CH)
    deg_e_pad = jnp.pad(degE[:, 0], (0, NEP - NE))
    xv2, _ = _sc_gather_scatter(xs, vtx2, edg2, deg_e_pad)
    xv = xv2[:, :N, :].transpose(1, 0, 2).reshape(N, D)
    wt = W.T
    a = jnp.float32(alpha)
    b = jnp.float32(beta)
    params = jnp.stack([1.0 - a, a, 1.0 - b, b])
    return _tc_tail(xv, X0, degV, wt, params)


# speedup vs baseline: 2.0597x; 2.0597x over previous
import functools

import jax
import jax.numpy as jnp
from jax import lax
from jax.experimental import pallas as pl
from jax.experimental.pallas import tpu as pltpu
from jax.experimental.pallas import tpu_sc as plsc

N = 10000
NNZ = 160000
NE = 5000
D = 256

NC = 2
NS = 16
LANES = 16

DH = D // NC
NEP = 6144
NP = 10240
EPC = NEP // NS
VPC = NP // NS
CH = 128
CPS = 80
NNZP = NS * CPS * CH
HB = CPS // 2
ST = 128
WO = 128


def _sc_gather_scatter(xs, vtx2, edg2, deg_e_pad):
    mesh = plsc.VectorSubcoreMesh(core_axis_name="c", subcore_axis_name="s")

    @functools.partial(
        pl.kernel,
        out_type=(
            jax.ShapeDtypeStruct((NC, NP, DH), jnp.float32),
            jax.ShapeDtypeStruct((NC, NEP, DH), jnp.float32),
        ),
        mesh=mesh,
        scratch_types=[
            pltpu.VMEM_SHARED((NP, DH), jnp.float32),
            pltpu.VMEM_SHARED((NEP,), jnp.float32),
            pltpu.VMEM((HB, CH), jnp.int32),
            pltpu.VMEM((HB, CH), jnp.int32),
            pltpu.VMEM((ST, DH), jnp.float32),
            pltpu.VMEM((ST, DH), jnp.float32),
            pltpu.VMEM((ST,), jnp.float32),
            pltpu.VMEM((EPC,), jnp.float32),
            pltpu.VMEM((ST,), jnp.float32),
            pltpu.VMEM((ST,), jnp.float32),
            pltpu.VMEM((ST,), jnp.float32),
            pltpu.SemaphoreType.DMA,
            pltpu.SemaphoreType.DMA,
            pltpu.SemaphoreType.DMA,
            pltpu.SemaphoreType.DMA,
            pltpu.SemaphoreType.DMA,
        ],
        compiler_params=pltpu.CompilerParams(needs_layout_passes=False),
    )
    def body(xs_hbm, vtx_hbm, edg_hbm, dege_hbm, out_hbm, xe_hbm,
             acc_sh, cnt_sh,
             vidx, eidx, rows0, rows1, ones, zcnt, cbuf, degb, scaleb,
             sem0, sem1, sem2, sem3, sem4):
        c = lax.axis_index("c")
        s = lax.axis_index("s")
        zeros16 = jnp.zeros((LANES,), jnp.float32)
        ones16 = jnp.ones((LANES,), jnp.float32)

        def zrow(r, _):
            for cc in range(DH // LANES):
                rows0[r, pl.ds(cc * LANES, LANES)] = zeros16
            return 0
        lax.fori_loop(0, ST, zrow, 0)
        for g in range(ST // LANES):
            ones[pl.ds(g * LANES, LANES)] = ones16
        for g in range(EPC // LANES):
            zcnt[pl.ds(g * LANES, LANES)] = zeros16

        e0 = s * EPC
        v0 = s * VPC
        b0 = s * CPS

        def run_phase(gather_tab, gsel, ssel, with_cnt):
            for h in range(2):
                pltpu.sync_copy(vtx_hbm.at[pl.ds(b0 + h * HB, HB)], vidx)
                pltpu.sync_copy(edg_hbm.at[pl.ds(b0 + h * HB, HB)], eidx)
                gi = vidx if gsel == "v" else eidx
                si = vidx if ssel == "v" else eidx

                def gath(j, buf, sem):
                    pltpu.async_copy(
                        gather_tab.at[c].at[gi.at[j]], buf, sem)

                def gath_wait(j, buf, sem):
                    pltpu.make_async_copy(
                        gather_tab.at[c].at[gi.at[j]], buf, sem).wait()

                def scat(j, buf, sem):
                    pltpu.async_copy(buf, acc_sh.at[si.at[j]], sem, add=True)
                    if with_cnt:
                        pltpu.async_copy(ones, cnt_sh.at[si.at[j]],
                                         sem4, add=True)

                def scat_wait(j, buf, sem):
                    pltpu.make_async_copy(buf, acc_sh.at[si.at[j]],
                                          sem).wait()

                gath(0, rows0, sem0)
                gath(1, rows1, sem1)

                def step(j2, _):
                    j = j2 * 2
                    gath_wait(j, rows0, sem0)
                    scat(j, rows0, sem2)
                    gath_wait(j + 1, rows1, sem1)
                    scat(j + 1, rows1, sem3)

                    @pl.when(j2 < HB // 2 - 1)
                    def _():
                        scat_wait(j, rows0, sem2)
                        gath(j + 2, rows0, sem0)
                        scat_wait(j + 1, rows1, sem3)
                        gath(j + 3, rows1, sem1)
                    return 0
                lax.fori_loop(0, HB // 2, step, 0)
                scat_wait(HB - 2, rows0, sem2)
                scat_wait(HB - 1, rows1, sem3)
                if with_cnt:
                    def drain(_, __):
                        pltpu.make_async_copy(ones, cnt_sh.at[si.at[0]],
                                              sem4).wait()
                        return 0
                    lax.fori_loop(0, HB, drain, 0)

        with jax.named_scope("zero_xe"):
            for j in range(EPC // ST):
                pltpu.sync_copy(rows0, acc_sh.at[pl.ds(e0 + j * ST, ST)])
            pltpu.sync_copy(zcnt, cnt_sh.at[pl.ds(e0, EPC)])
            plsc.subcore_barrier()

        with jax.named_scope("phase1"):
            run_phase(xs_hbm, "v", "e", True)
            plsc.subcore_barrier()

        with jax.named_scope("rescale"):
          for j in range(EPC // ST):
            r0 = e0 + j * ST
            pltpu.sync_copy(acc_sh.at[pl.ds(r0, ST)], rows0)
            pltpu.sync_copy(cnt_sh.at[pl.ds(r0, ST)], cbuf)
            pltpu.sync_copy(dege_hbm.at[pl.ds(r0, ST)], degb)
            for g in range(ST // LANES):
                sl = pl.ds(g * LANES, LANES)
                scaleb[sl] = degb[sl] / jnp.maximum(cbuf[sl], 1.0)

            def scalerow(r, _):
                bs = plsc.load_gather(
                    scaleb, [jnp.full((LANES,), r, jnp.int32)])
                for cc in range(DH // LANES):
                    sl = pl.ds(cc * LANES, LANES)
                    rows0[r, sl] = rows0[r, sl] * bs
                return 0
            lax.fori_loop(0, ST, scalerow, 0)
            pltpu.sync_copy(rows0, xe_hbm.at[c].at[pl.ds(r0, ST)])
        plsc.subcore_barrier()

        with jax.named_scope("rezero"):
          def zrow2(r, _):
            for cc in range(DH // LANES):
                rows0[r, pl.ds(cc * LANES, LANES)] = zeros16
            return 0
          lax.fori_loop(0, ST, zrow2, 0)
          for j in range(VPC // WO):
            pltpu.sync_copy(rows0, acc_sh.at[pl.ds(v0 + j * WO, WO)])
          plsc.subcore_barrier()

        with jax.named_scope("phase2"):
            run_phase(xe_hbm, "e", "v", False)
            plsc.subcore_barrier()

        with jax.named_scope("writeout"):
          for j in range(VPC // WO):
            w0 = v0 + j * WO
            pltpu.sync_copy(acc_sh.at[pl.ds(w0, WO)], rows0)
            pltpu.sync_copy(rows0, out_hbm.at[c].at[pl.ds(w0, WO)])

    return body(xs, vtx2, edg2, deg_e_pad)


def _tc_tail_body(xv_ref, x0_ref, degv_ref, wt_ref, p_ref, out_ref):
    degv = degv_ref[...]
    xv = xv_ref[...] * degv
    n = jnp.sum(xv * xv, axis=1, keepdims=True)
    rn = jnp.sqrt(n)
    scale = jnp.where(rn > 0, 1.0 / rn, 0.0)
    xi = p_ref[0] * (xv * scale) + p_ref[1] * x0_ref[...]
    mm = jnp.dot(xi, wt_ref[...], preferred_element_type=jnp.float32)
    out_ref[...] = p_ref[2] * xi + p_ref[3] * mm


def _tc_tail(xv, x0, degv, wt, params):
    blk = 1000
    grid = (N // blk,)
    return pl.pallas_call(
        _tc_tail_body,
        grid=grid,
        in_specs=[
            pl.BlockSpec((blk, D), lambda i: (i, 0)),
            pl.BlockSpec((blk, D), lambda i: (i, 0)),
            pl.BlockSpec((blk, 1), lambda i: (i, 0)),
            pl.BlockSpec((D, D), lambda i: (0, 0)),
            pl.BlockSpec(memory_space=pltpu.SMEM),
        ],
        out_specs=pl.BlockSpec((blk, D), lambda i: (i, 0)),
        out_shape=jax.ShapeDtypeStruct((N, D), jnp.float32),
    )(xv, x0, degv, wt, params)


def kernel(X, vertex, edges, alpha, beta, X0, degE, degV, W):
    xs = jnp.pad(X.reshape(N, NC, DH).transpose(1, 0, 2),
                 ((0, 0), (0, NP - N), (0, 0)))
    npad = NNZP - NNZ
    pad_v = N + (jnp.arange(npad, dtype=jnp.int32) % (NP - N))
    pad_e = NE + (jnp.arange(npad, dtype=jnp.int32) % (NEP - NE))
    vtx2 = jnp.concatenate([vertex, pad_v]).reshape(NS * CPS, CH)
    edg2 = jnp.concatenate([edges, pad_e]).reshape(NS * CPS, CH)
    deg_e_pad = jnp.pad(degE[:, 0], (0, NEP - NE))
    xv2, _ = _sc_gather_scatter(xs, vtx2, edg2, deg_e_pad)
    xv = xv2[:, :N, :].transpose(1, 0, 2).reshape(N, D)
    wt = W.T
    a = jnp.float32(alpha)
    b = jnp.float32(beta)
    params = jnp.stack([1.0 - a, a, 1.0 - b, b])
    return _tc_tail(xv, X0, degV, wt, params)


# speedup vs baseline: 2.1379x; 1.0380x over previous
import functools

import jax
import jax.numpy as jnp
from jax import lax
from jax.experimental import pallas as pl
from jax.experimental.pallas import tpu as pltpu
from jax.experimental.pallas import tpu_sc as plsc

N = 10000
NNZ = 160000
NE = 5000
D = 256

NC = 2
NS = 16
LANES = 16

DH = D // NC
NEP = 6144
NP = 10240
EPC = NEP // NS
VPC = NP // NS
CH = 125
CPS = NNZ // NS // CH
HB = CPS // 2
ST = 128
WO = 128


def _sc_gather_scatter(xs, vtx2, edg2, deg_e_pad):
    mesh = plsc.VectorSubcoreMesh(core_axis_name="c", subcore_axis_name="s")

    @functools.partial(
        pl.kernel,
        out_type=(
            jax.ShapeDtypeStruct((NP, NC, DH), jnp.float32),
            jax.ShapeDtypeStruct((NC, NEP, DH), jnp.float32),
        ),
        mesh=mesh,
        scratch_types=[
            pltpu.VMEM_SHARED((NP, DH), jnp.float32),
            pltpu.VMEM_SHARED((NEP,), jnp.float32),
            pltpu.VMEM((HB, CH), jnp.int32),
            pltpu.VMEM((HB, CH), jnp.int32),
            pltpu.VMEM((ST, DH), jnp.float32),
            pltpu.VMEM((ST, DH), jnp.float32),
            pltpu.VMEM((ST,), jnp.float32),
            pltpu.VMEM((EPC,), jnp.float32),
            pltpu.VMEM((ST,), jnp.float32),
            pltpu.VMEM((ST,), jnp.float32),
            pltpu.VMEM((ST,), jnp.float32),
            pltpu.SemaphoreType.DMA,
            pltpu.SemaphoreType.DMA,
            pltpu.SemaphoreType.DMA,
            pltpu.SemaphoreType.DMA,
            pltpu.SemaphoreType.DMA,
        ],
        compiler_params=pltpu.CompilerParams(needs_layout_passes=False),
    )
    def body(xs_hbm, vtx_hbm, edg_hbm, dege_hbm, out_hbm, xe_hbm,
             acc_sh, cnt_sh,
             vidx, eidx, rows0, rows1, ones, zcnt, cbuf, degb, scaleb,
             sem0, sem1, sem2, sem3, sem4):
        c = lax.axis_index("c")
        s = lax.axis_index("s")
        zeros16 = jnp.zeros((LANES,), jnp.float32)
        ones16 = jnp.ones((LANES,), jnp.float32)

        def zrow(r, _):
            for cc in range(DH // LANES):
                rows0[r, pl.ds(cc * LANES, LANES)] = zeros16
            return 0
        lax.fori_loop(0, ST, zrow, 0)
        for g in range(ST // LANES):
            ones[pl.ds(g * LANES, LANES)] = ones16
        for g in range(EPC // LANES):
            zcnt[pl.ds(g * LANES, LANES)] = zeros16

        e0 = s * EPC
        v0 = s * VPC
        b0 = s * CPS

        def run_phase(mk_src, gsel, ssel, with_cnt):
            for h in range(2):
                pltpu.sync_copy(vtx_hbm.at[pl.ds(b0 + h * HB, HB)], vidx)
                pltpu.sync_copy(edg_hbm.at[pl.ds(b0 + h * HB, HB)], eidx)
                gi = vidx if gsel == "v" else eidx
                si = vidx if ssel == "v" else eidx
                rsl = pl.ds(0, CH)

                def gath(j, buf, sem):
                    pltpu.async_copy(mk_src(gi.at[j]), buf.at[rsl], sem)

                def gath_wait(j, buf, sem):
                    pltpu.make_async_copy(mk_src(gi.at[j]), buf.at[rsl],
                                          sem).wait()

                def scat(j, buf, sem):
                    pltpu.async_copy(buf.at[rsl], acc_sh.at[si.at[j]], sem,
                                     add=True)
                    if with_cnt:
                        pltpu.async_copy(ones.at[rsl], cnt_sh.at[si.at[j]],
                                         sem4, add=True)

                def scat_wait(j, buf, sem):
                    pltpu.make_async_copy(buf.at[rsl], acc_sh.at[si.at[j]],
                                          sem).wait()

                gath(0, rows0, sem0)
                gath(1, rows1, sem1)

                def step(j2, _):
                    j = j2 * 2
                    gath_wait(j, rows0, sem0)
                    scat(j, rows0, sem2)
                    gath_wait(j + 1, rows1, sem1)
                    scat(j + 1, rows1, sem3)

                    @pl.when(j2 < HB // 2 - 1)
                    def _():
                        scat_wait(j, rows0, sem2)
                        gath(j + 2, rows0, sem0)
                        scat_wait(j + 1, rows1, sem3)
                        gath(j + 3, rows1, sem1)
                    return 0
                lax.fori_loop(0, HB // 2, step, 0)
                scat_wait(HB - 2, rows0, sem2)
                scat_wait(HB - 1, rows1, sem3)
                if with_cnt:
                    def drain(_, __):
                        pltpu.make_async_copy(ones.at[rsl],
                                              cnt_sh.at[si.at[0]],
                                              sem4).wait()
                        return 0
                    lax.fori_loop(0, HB, drain, 0)

        with jax.named_scope("zero_xe"):
            for j in range(EPC // ST):
                pltpu.sync_copy(rows0, acc_sh.at[pl.ds(e0 + j * ST, ST)])
            pltpu.sync_copy(zcnt, cnt_sh.at[pl.ds(e0, EPC)])
            plsc.subcore_barrier()

        with jax.named_scope("phase1"):
            coff = pl.multiple_of(c * DH, DH)
            run_phase(lambda idx: xs_hbm.at[:, pl.ds(coff, DH)].at[idx],
                      "v", "e", True)
            plsc.subcore_barrier()

        with jax.named_scope("rescale"):
          for j in range(EPC // ST):
            r0 = e0 + j * ST
            pltpu.sync_copy(acc_sh.at[pl.ds(r0, ST)], rows0)
            pltpu.sync_copy(cnt_sh.at[pl.ds(r0, ST)], cbuf)
            pltpu.sync_copy(dege_hbm.at[pl.ds(r0, ST)], degb)
            for g in range(ST // LANES):
                sl = pl.ds(g * LANES, LANES)
                scaleb[sl] = degb[sl] / jnp.maximum(cbuf[sl], 1.0)

            def scalerow(r, _):
                bs = plsc.load_gather(
                    scaleb, [jnp.full((LANES,), r, jnp.int32)])
                for cc in range(DH // LANES):
                    sl = pl.ds(cc * LANES, LANES)
                    rows0[r, sl] = rows0[r, sl] * bs
                return 0
            lax.fori_loop(0, ST, scalerow, 0)
            pltpu.sync_copy(rows0, xe_hbm.at[c].at[pl.ds(r0, ST)])
        plsc.subcore_barrier()

        with jax.named_scope("rezero"):
          def zrow2(r, _):
            for cc in range(DH // LANES):
                rows0[r, pl.ds(cc * LANES, LANES)] = zeros16
            return 0
          lax.fori_loop(0, ST, zrow2, 0)
          for j in range(VPC // WO):
            pltpu.sync_copy(rows0, acc_sh.at[pl.ds(v0 + j * WO, WO)])
          plsc.subcore_barrier()

        with jax.named_scope("phase2"):
            run_phase(lambda idx: xe_hbm.at[c].at[idx], "e", "v", False)
            plsc.subcore_barrier()

        with jax.named_scope("writeout"):
          for j in range(VPC // WO):
            w0 = v0 + j * WO
            pltpu.sync_copy(acc_sh.at[pl.ds(w0, WO)], rows0)
            pltpu.sync_copy(rows0, out_hbm.at[pl.ds(w0, WO), c])

    return body(xs, vtx2, edg2, deg_e_pad)


def _tc_tail_body(xv_ref, x0_ref, degv_ref, wt_ref, p_ref, out_ref):
    degv = degv_ref[...]
    xv = xv_ref[...] * degv
    n = jnp.sum(xv * xv, axis=1, keepdims=True)
    rn = jnp.sqrt(n)
    scale = jnp.where(rn > 0, 1.0 / rn, 0.0)
    xi = p_ref[0] * (xv * scale) + p_ref[1] * x0_ref[...]
    mm = jnp.dot(xi, wt_ref[...], preferred_element_type=jnp.float32)
    out_ref[...] = p_ref[2] * xi + p_ref[3] * mm


def _tc_tail(xv, x0, degv, wt, params):
    blk = 1000
    grid = (N // blk,)
    return pl.pallas_call(
        _tc_tail_body,
        grid=grid,
        in_specs=[
            pl.BlockSpec((blk, D), lambda i: (i, 0)),
            pl.BlockSpec((blk, D), lambda i: (i, 0)),
            pl.BlockSpec((blk, 1), lambda i: (i, 0)),
            pl.BlockSpec((D, D), lambda i: (0, 0)),
            pl.BlockSpec(memory_space=pltpu.SMEM),
        ],
        out_specs=pl.BlockSpec((blk, D), lambda i: (i, 0)),
        out_shape=jax.ShapeDtypeStruct((N, D), jnp.float32),
    )(xv, x0, degv, wt, params)


def kernel(X, vertex, edges, alpha, beta, X0, degE, degV, W):
    xs = X
    vtx2 = vertex.reshape(NS * CPS, CH)
    edg2 = edges.reshape(NS * CPS, CH)
    deg_e_pad = jnp.pad(degE[:, 0], (0, NEP - NE))
    xv2, _ = _sc_gather_scatter(xs, vtx2, edg2, deg_e_pad)
    xv = xv2.reshape(NP, D)[:N]
    wt = W.T
    a = jnp.float32(alpha)
    b = jnp.float32(beta)
    params = jnp.stack([1.0 - a, a, 1.0 - b, b])
    return _tc_tail(xv, X0, degV, wt, params)
